# baseline (speedup 1.0000x reference)
import functools
import itertools

import jax
import jax.numpy as jnp
import numpy as np
from jax import lax
from jax.experimental import pallas as pl
from jax.experimental.pallas import tpu as pltpu
from jax.experimental.pallas import tpu_sc as plsc

_B = 4096
_F = 26
_D0 = 16
_EMB = 32
_NB = 1000000
_PAIRS = np.array(list(itertools.combinations(range(_F), 2)), dtype=np.int32)
_K = _PAIRS.shape[0]

_IK = _PAIRS[:, 0]
_JK = _PAIRS[:, 1]
_CK = (_IK.astype(np.int32) * 1822 + _JK.astype(np.int32) * 6649)

_F2I = np.zeros((_F, _F - 1), dtype=np.int32)
_cnt = np.zeros(_F, dtype=np.int32)
for _k, (_i, _j) in enumerate(_PAIRS):
    _F2I[_i, _cnt[_i]] = _k; _cnt[_i] += 1
    _F2I[_j, _cnt[_j]] = _k; _cnt[_j] += 1

_S_T = np.zeros((_F, _K), dtype=np.float32)
_S_T[_IK, np.arange(_K)] = 1.0
_S_T[_JK, np.arange(_K)] = 1.0

_GFW = np.zeros((_K, _F * (_F - 1)), dtype=np.float32)
_GFW[_F2I.reshape(-1), np.arange(_F * (_F - 1))] = 1.0

_NC = 2
_NS = 16
_NW = _NC * _NS
_BPW = _B // _NW
assert _BPW * _NW == _B


def _sc_gather(x, ik, jk, ck, codebook):
    mesh = plsc.VectorSubcoreMesh(core_axis_name="c", subcore_axis_name="s")

    @functools.partial(
        pl.kernel,
        out_type=jax.ShapeDtypeStruct((_K * _B, _EMB), jnp.float32),
        mesh=mesh,
        scratch_types=[
            pltpu.VMEM((_BPW, _F), jnp.int32),
            pltpu.VMEM((_K,), jnp.int32),
            pltpu.VMEM((_K,), jnp.int32),
            pltpu.VMEM((_K,), jnp.int32),
            pltpu.VMEM((2, _BPW), jnp.int32),
            pltpu.VMEM((2, _BPW, _EMB), jnp.float32),
            pltpu.SemaphoreType.DMA,
        ],
        compiler_params=pltpu.CompilerParams(
            use_tc_tiling_on_sc=False, needs_layout_passes=False
        ),
    )
    def gather_kernel(x_hbm, ik_hbm, jk_hbm, ck_hbm, table_hbm, out_hbm,
                      x_v, ik_v, jk_v, ck_v, idx_v, rows_v, gsem):
        wid = lax.axis_index("s") * _NC + lax.axis_index("c")
        bbase = pl.multiple_of(wid * _BPW, _BPW)
        pltpu.sync_copy(x_hbm.at[pl.ds(bbase, _BPW), :], x_v)
        pltpu.sync_copy(ik_hbm, ik_v)
        pltpu.sync_copy(jk_hbm, jk_v)
        pltpu.sync_copy(ck_hbm, ck_v)

        lanes = jnp.arange(16, dtype=jnp.int32)

        def hash_chunk(c, slot):
            cvec = jnp.broadcast_to(c, (16,)).astype(jnp.int32)
            ikvec = plsc.load_gather(ik_v, [cvec])
            jkvec = plsc.load_gather(jk_v, [cvec])
            ckvec = plsc.load_gather(ck_v, [cvec])
            for li in range(_BPW // 16):
                bvec = lanes + (li * 16)
                xi = plsc.load_gather(x_v, [bvec, ikvec])
                xj = plsc.load_gather(x_v, [bvec, jkvec])
                h = xi * 40503 + xj * 7744 + ckvec
                r = lax.rem(h, _NB)
                r = jnp.where(r < 0, r + _NB, r)
                idx_v[slot, pl.ds(li * 16, 16)] = r

        def start(slot):
            pltpu.async_copy(
                table_hbm.at[idx_v.at[slot]], rows_v.at[slot], gsem
            )

        def body(c, carry):
            slot = lax.rem(c, 2)
            pslot = lax.rem(c + 1, 2)

            @pl.when(c < _K)
            def _launch():
                hash_chunk(c, slot)
                start(slot)

            @pl.when(c >= 1)
            def _drain():
                pltpu.make_async_copy(
                    table_hbm.at[idx_v.at[pslot]], rows_v.at[pslot], gsem
                ).wait()
                orow = pl.multiple_of(
                    (c - 1) * _B + wid * _BPW, _BPW
                )
                pltpu.sync_copy(
                    rows_v.at[pslot], out_hbm.at[pl.ds(orow, _BPW), :]
                )

            return carry

        lax.fori_loop(0, _K + 1, body, 0)

    return gather_kernel(x, ik, jk, ck, codebook)


_SENET_BT = 256


def _senet_body(z_ref, w1_ref, w2_ref, gfw_ref, wt_ref, fw_ref):
    z = z_ref[...]
    t1 = jnp.dot(z, w1_ref[...], preferred_element_type=jnp.float32)
    w = jnp.dot(t1, w2_ref[...], preferred_element_type=jnp.float32)
    wt = lax.dot_general(
        w2_ref[...], t1, (((0,), (1,)), ((), ())),
        preferred_element_type=jnp.float32,
    )
    wt_ref[...] = wt
    fw_ref[...] = jnp.dot(w, gfw_ref[...], preferred_element_type=jnp.float32)


def _senet(z, w1, w2, gfw):
    nt = _B // _SENET_BT
    return pl.pallas_call(
        _senet_body,
        grid=(nt,),
        in_specs=[
            pl.BlockSpec((_SENET_BT, _F * _D0), lambda i: (i, 0)),
            pl.BlockSpec((_F * _D0, _F * _D0), lambda i: (0, 0)),
            pl.BlockSpec((_F * _D0, _K), lambda i: (0, 0)),
            pl.BlockSpec((_K, _F * (_F - 1)), lambda i: (0, 0)),
        ],
        out_specs=[
            pl.BlockSpec((_K, _SENET_BT), lambda i: (0, i)),
            pl.BlockSpec((_SENET_BT, _F * (_F - 1)), lambda i: (i, 0)),
        ],
        out_shape=[
            jax.ShapeDtypeStruct((_K, _B), jnp.float32),
            jax.ShapeDtypeStruct((_B, _F * (_F - 1)), jnp.float32),
        ],
    )(z, w1, w2, gfw)


_MERGE_BT = 128


_KC = 65
_BT4 = _MERGE_BT // 4


def _merge_body(g_ref, wt_ref, st_ref, out_ref):
    acc = jnp.zeros((_F, _MERGE_BT * _EMB), jnp.float32)
    for c in range(_K // _KC):
        g = g_ref[pl.ds(c * _KC, _KC)]
        w = wt_ref[pl.ds(c * _KC, _KC)]
        w4 = jnp.broadcast_to(
            w.reshape(_KC, _BT4, 4)[:, :, :, None], (_KC, _BT4, 4, _EMB)
        ).reshape(_KC, _BT4, 4 * _EMB)
        wg = (g * w4).reshape(_KC, _MERGE_BT * _EMB)
        st = st_ref[:, pl.ds(c * _KC, _KC)]
        acc = acc + jnp.dot(
            st.astype(jnp.bfloat16),
            wg.astype(jnp.bfloat16),
            preferred_element_type=jnp.float32,
        )
    out_ref[...] = acc.reshape(_F, _BT4, 4 * _EMB)


def _merge(g4, wt, st):
    nt = _B // _MERGE_BT
    return pl.pallas_call(
        _merge_body,
        grid=(nt,),
        in_specs=[
            pl.BlockSpec((_K, _BT4, 4 * _EMB), lambda j: (0, j, 0)),
            pl.BlockSpec((_K, _MERGE_BT), lambda j: (0, j)),
            pl.BlockSpec((_F, _K), lambda j: (0, 0)),
        ],
        out_specs=pl.BlockSpec((_F, _BT4, 4 * _EMB), lambda j: (0, j, 0)),
        out_shape=jax.ShapeDtypeStruct((_F, _B // 4, 4 * _EMB), jnp.float32),
    )(g4, wt, st)


def kernel(placeholder_inputs, origin_embeddings, codebook, senet_w1, senet_w2):
    g = _sc_gather(
        placeholder_inputs,
        jnp.asarray(_IK),
        jnp.asarray(_JK),
        jnp.asarray(_CK),
        codebook,
    )
    g4 = g.reshape(_K, _B // 4, 4 * _EMB)
    z = origin_embeddings.reshape(_B, _F * _D0)
    wt, fw = _senet(z, senet_w1, senet_w2, jnp.asarray(_GFW))
    out_t = _merge(g4, wt, jnp.asarray(_S_T))
    outputs = jnp.swapaxes(out_t.reshape(_F, _B, _EMB), 0, 1)
    field_weights = fw.reshape(_B, _F, _F - 1, 1)
    return outputs, field_weights


# speedup vs baseline: 1.6668x; 1.6668x over previous
import functools
import itertools

import jax
import jax.numpy as jnp
import numpy as np
from jax import lax
from jax.experimental import pallas as pl
from jax.experimental.pallas import tpu as pltpu
from jax.experimental.pallas import tpu_sc as plsc

_B = 4096
_F = 26
_D0 = 16
_EMB = 32
_NB = 1000000
_PAIRS = np.array(list(itertools.combinations(range(_F), 2)), dtype=np.int32)
_K = _PAIRS.shape[0]

_IK = _PAIRS[:, 0]
_JK = _PAIRS[:, 1]
_CK = (_IK.astype(np.int32) * 1822 + _JK.astype(np.int32) * 6649)

_F2I = np.zeros((_F, _F - 1), dtype=np.int32)
_cnt = np.zeros(_F, dtype=np.int32)
for _k, (_i, _j) in enumerate(_PAIRS):
    _F2I[_i, _cnt[_i]] = _k; _cnt[_i] += 1
    _F2I[_j, _cnt[_j]] = _k; _cnt[_j] += 1

_S_T = np.zeros((_F, _K), dtype=np.float32)
_S_T[_IK, np.arange(_K)] = 1.0
_S_T[_JK, np.arange(_K)] = 1.0

_GFW = np.zeros((_K, _F * (_F - 1)), dtype=np.float32)
_GFW[_F2I.reshape(-1), np.arange(_F * (_F - 1))] = 1.0

_E4 = np.zeros((128, 128 * 32), dtype=np.float32)
for _b in range(128):
    _E4[_b, (_b // 4) * 128 + (_b % 4) * 32 : (_b // 4) * 128 + (_b % 4) * 32 + 32] = 1.0

_NC = 2
_NS = 16
_NW = _NC * _NS
_BPW = _B // _NW
assert _BPW * _NW == _B


def _sc_gather(x, ik, jk, ck, codebook):
    mesh = plsc.VectorSubcoreMesh(core_axis_name="c", subcore_axis_name="s")

    @functools.partial(
        pl.kernel,
        out_type=jax.ShapeDtypeStruct((_K * _B, _EMB), jnp.float32),
        mesh=mesh,
        scratch_types=[
            pltpu.VMEM((_BPW, _F), jnp.int32),
            pltpu.VMEM((_K,), jnp.int32),
            pltpu.VMEM((_K,), jnp.int32),
            pltpu.VMEM((_K,), jnp.int32),
            pltpu.VMEM((2, _BPW), jnp.int32),
            pltpu.VMEM((2, _BPW, _EMB), jnp.float32),
            pltpu.SemaphoreType.DMA,
        ],
        compiler_params=pltpu.CompilerParams(
            use_tc_tiling_on_sc=False, needs_layout_passes=False
        ),
    )
    def gather_kernel(x_hbm, ik_hbm, jk_hbm, ck_hbm, table_hbm, out_hbm,
                      x_v, ik_v, jk_v, ck_v, idx_v, rows_v, gsem):
        wid = lax.axis_index("s") * _NC + lax.axis_index("c")
        bbase = pl.multiple_of(wid * _BPW, _BPW)
        pltpu.sync_copy(x_hbm.at[pl.ds(bbase, _BPW), :], x_v)
        pltpu.sync_copy(ik_hbm, ik_v)
        pltpu.sync_copy(jk_hbm, jk_v)
        pltpu.sync_copy(ck_hbm, ck_v)

        lanes = jnp.arange(16, dtype=jnp.int32)

        def hash_chunk(c, slot):
            cvec = jnp.broadcast_to(c, (16,)).astype(jnp.int32)
            ikvec = plsc.load_gather(ik_v, [cvec])
            jkvec = plsc.load_gather(jk_v, [cvec])
            ckvec = plsc.load_gather(ck_v, [cvec])
            for li in range(_BPW // 16):
                bvec = lanes + (li * 16)
                xi = plsc.load_gather(x_v, [bvec, ikvec])
                xj = plsc.load_gather(x_v, [bvec, jkvec])
                h = xi * 40503 + xj * 7744 + ckvec
                r = lax.rem(h, _NB)
                r = jnp.where(r < 0, r + _NB, r)
                idx_v[slot, pl.ds(li * 16, 16)] = r

        def start(slot):
            pltpu.async_copy(
                table_hbm.at[idx_v.at[slot]], rows_v.at[slot], gsem
            )

        def body(c, carry):
            slot = lax.rem(c, 2)
            pslot = lax.rem(c + 1, 2)

            @pl.when(c < _K)
            def _launch():
                hash_chunk(c, slot)
                start(slot)

            @pl.when(c >= 1)
            def _drain():
                pltpu.make_async_copy(
                    table_hbm.at[idx_v.at[pslot]], rows_v.at[pslot], gsem
                ).wait()
                orow = pl.multiple_of(
                    (c - 1) * _B + wid * _BPW, _BPW
                )
                pltpu.sync_copy(
                    rows_v.at[pslot], out_hbm.at[pl.ds(orow, _BPW), :]
                )

            return carry

        lax.fori_loop(0, _K + 1, body, 0)

    return gather_kernel(x, ik, jk, ck, codebook)


_SENET_BT = 256


def _senet_body(z_ref, w1_ref, w2_ref, gfw_ref, wt_ref, fw_ref):
    z = z_ref[...]
    t1 = jnp.dot(z, w1_ref[...], preferred_element_type=jnp.float32)
    w = jnp.dot(t1, w2_ref[...], preferred_element_type=jnp.float32)
    wt = lax.dot_general(
        w2_ref[...], t1, (((0,), (1,)), ((), ())),
        preferred_element_type=jnp.float32,
    )
    wt_ref[...] = wt
    fw_ref[...] = jnp.dot(w, gfw_ref[...], preferred_element_type=jnp.float32)


def _senet(z, w1, w2, gfw):
    nt = _B // _SENET_BT
    return pl.pallas_call(
        _senet_body,
        grid=(nt,),
        in_specs=[
            pl.BlockSpec((_SENET_BT, _F * _D0), lambda i: (i, 0)),
            pl.BlockSpec((_F * _D0, _F * _D0), lambda i: (0, 0)),
            pl.BlockSpec((_F * _D0, _K), lambda i: (0, 0)),
            pl.BlockSpec((_K, _F * (_F - 1)), lambda i: (0, 0)),
        ],
        out_specs=[
            pl.BlockSpec((_K, _SENET_BT), lambda i: (0, i)),
            pl.BlockSpec((_SENET_BT, _F * (_F - 1)), lambda i: (i, 0)),
        ],
        out_shape=[
            jax.ShapeDtypeStruct((_K, _B), jnp.float32),
            jax.ShapeDtypeStruct((_B, _F * (_F - 1)), jnp.float32),
        ],
    )(z, w1, w2, gfw)


_MERGE_BT = 128


_KC = 65
_BT4 = _MERGE_BT // 4


def _merge_body(g_ref, wt_ref, st_ref, e4_ref, out_ref):
    acc = jnp.zeros((_F, _MERGE_BT * _EMB), jnp.float32)
    for c in range(_K // _KC):
        g = g_ref[pl.ds(c * _KC, _KC)]
        w = wt_ref[pl.ds(c * _KC, _KC)]
        w4 = jnp.dot(w, e4_ref[...], preferred_element_type=jnp.float32)
        wg = g.reshape(_KC, _MERGE_BT * _EMB) * w4
        st = st_ref[:, pl.ds(c * _KC, _KC)]
        acc = acc + jnp.dot(
            st.astype(jnp.bfloat16),
            wg.astype(jnp.bfloat16),
            preferred_element_type=jnp.float32,
        )
    out_ref[...] = acc.reshape(_F, _BT4, 4 * _EMB)


def _merge(g4, wt, st, e4):
    nt = _B // _MERGE_BT
    return pl.pallas_call(
        _merge_body,
        grid=(nt,),
        in_specs=[
            pl.BlockSpec((_K, _BT4, 4 * _EMB), lambda j: (0, j, 0)),
            pl.BlockSpec((_K, _MERGE_BT), lambda j: (0, j)),
            pl.BlockSpec((_F, _K), lambda j: (0, 0)),
            pl.BlockSpec((_MERGE_BT, _MERGE_BT * _EMB), lambda j: (0, 0)),
        ],
        out_specs=pl.BlockSpec((_F, _BT4, 4 * _EMB), lambda j: (0, j, 0)),
        out_shape=jax.ShapeDtypeStruct((_F, _B // 4, 4 * _EMB), jnp.float32),
    )(g4, wt, st, e4)


def kernel(placeholder_inputs, origin_embeddings, codebook, senet_w1, senet_w2):
    g = _sc_gather(
        placeholder_inputs,
        jnp.asarray(_IK),
        jnp.asarray(_JK),
        jnp.asarray(_CK),
        codebook,
    )
    g4 = g.reshape(_K, _B // 4, 4 * _EMB)
    z = origin_embeddings.reshape(_B, _F * _D0)
    wt, fw = _senet(z, senet_w1, senet_w2, jnp.asarray(_GFW))
    out_t = _merge(g4, wt, jnp.asarray(_S_T), jnp.asarray(_E4))
    outputs = jnp.swapaxes(out_t.reshape(_F, _B, _EMB), 0, 1)
    field_weights = fw.reshape(_B, _F, _F - 1, 1)
    return outputs, field_weights


# speedup vs baseline: 1.7202x; 1.0321x over previous
import functools
import itertools

import jax
import jax.numpy as jnp
import numpy as np
from jax import lax
from jax.experimental import pallas as pl
from jax.experimental.pallas import tpu as pltpu
from jax.experimental.pallas import tpu_sc as plsc

_B = 4096
_F = 26
_D0 = 16
_EMB = 32
_NB = 1000000
_PAIRS = np.array(list(itertools.combinations(range(_F), 2)), dtype=np.int32)
_K = _PAIRS.shape[0]

_IK = _PAIRS[:, 0]
_JK = _PAIRS[:, 1]
_CK = (_IK.astype(np.int32) * 1822 + _JK.astype(np.int32) * 6649)

_F2I = np.zeros((_F, _F - 1), dtype=np.int32)
_cnt = np.zeros(_F, dtype=np.int32)
for _k, (_i, _j) in enumerate(_PAIRS):
    _F2I[_i, _cnt[_i]] = _k; _cnt[_i] += 1
    _F2I[_j, _cnt[_j]] = _k; _cnt[_j] += 1

_S_T = np.zeros((_F, _K), dtype=np.float32)
_S_T[_IK, np.arange(_K)] = 1.0
_S_T[_JK, np.arange(_K)] = 1.0

_GFW = np.zeros((_K, _F * (_F - 1)), dtype=np.float32)
_GFW[_F2I.reshape(-1), np.arange(_F * (_F - 1))] = 1.0

_E4 = np.zeros((128, 128 * 32), dtype=np.float32)
for _b in range(128):
    _E4[_b, (_b // 4) * 128 + (_b % 4) * 32 : (_b // 4) * 128 + (_b % 4) * 32 + 32] = 1.0

_NC = 2
_NS = 16
_NW = _NC * _NS
_BPW = _B // _NW
assert _BPW * _NW == _B


def _sc_gather(x, ik, jk, ck, codebook):
    mesh = plsc.VectorSubcoreMesh(core_axis_name="c", subcore_axis_name="s")

    @functools.partial(
        pl.kernel,
        out_type=jax.ShapeDtypeStruct((_K * _B, _EMB), jnp.float32),
        mesh=mesh,
        scratch_types=[
            pltpu.VMEM((_BPW, _F), jnp.int32),
            pltpu.VMEM((_K,), jnp.int32),
            pltpu.VMEM((_K,), jnp.int32),
            pltpu.VMEM((_K,), jnp.int32),
            pltpu.VMEM((2, _BPW), jnp.int32),
            pltpu.VMEM((2, _BPW, _EMB), jnp.float32),
            pltpu.SemaphoreType.DMA,
            pltpu.SemaphoreType.DMA,
        ],
        compiler_params=pltpu.CompilerParams(
            use_tc_tiling_on_sc=False, needs_layout_passes=False
        ),
    )
    def gather_kernel(x_hbm, ik_hbm, jk_hbm, ck_hbm, table_hbm, out_hbm,
                      x_v, ik_v, jk_v, ck_v, idx_v, rows_v, gsem, wsem):
        wid = lax.axis_index("s") * _NC + lax.axis_index("c")
        bbase = pl.multiple_of(wid * _BPW, _BPW)
        pltpu.sync_copy(x_hbm.at[pl.ds(bbase, _BPW), :], x_v)
        pltpu.sync_copy(ik_hbm, ik_v)
        pltpu.sync_copy(jk_hbm, jk_v)
        pltpu.sync_copy(ck_hbm, ck_v)

        lanes = jnp.arange(16, dtype=jnp.int32)

        def hash_chunk(c, slot):
            cvec = jnp.broadcast_to(c, (16,)).astype(jnp.int32)
            ikvec = plsc.load_gather(ik_v, [cvec])
            jkvec = plsc.load_gather(jk_v, [cvec])
            ckvec = plsc.load_gather(ck_v, [cvec])
            for li in range(_BPW // 16):
                bvec = lanes + (li * 16)
                xi = plsc.load_gather(x_v, [bvec, ikvec])
                xj = plsc.load_gather(x_v, [bvec, jkvec])
                h = xi * 40503 + xj * 7744 + ckvec
                r = lax.rem(h, _NB)
                r = jnp.where(r < 0, r + _NB, r)
                idx_v[slot, pl.ds(li * 16, 16)] = r

        def start(slot):
            pltpu.async_copy(
                table_hbm.at[idx_v.at[slot]], rows_v.at[slot], gsem
            )

        def out_at(c):
            orow = pl.multiple_of(c * _B + wid * _BPW, _BPW)
            return out_hbm.at[pl.ds(orow, _BPW), :]

        def body(c, carry):
            slot = lax.rem(c, 2)
            pslot = lax.rem(c + 1, 2)

            @pl.when(c < _K)
            def _launch():
                hash_chunk(c, slot)

                @pl.when(c >= 2)
                def _drain_write():
                    pltpu.make_async_copy(
                        rows_v.at[slot], out_at(c - 2), wsem
                    ).wait()

                start(slot)

            @pl.when(c >= 1)
            def _emit():
                pltpu.make_async_copy(
                    table_hbm.at[idx_v.at[pslot]], rows_v.at[pslot], gsem
                ).wait()
                pltpu.async_copy(rows_v.at[pslot], out_at(c - 1), wsem)

            return carry

        lax.fori_loop(0, _K + 1, body, 0)

        pltpu.make_async_copy(rows_v.at[0], out_at(_K - 1), wsem).wait()
        pltpu.make_async_copy(rows_v.at[1], out_at(_K - 2), wsem).wait()

    return gather_kernel(x, ik, jk, ck, codebook)


_SENET_BT = 256


def _senet_body(z_ref, w1_ref, w2_ref, gfw_ref, wt_ref, fw_ref):
    z = z_ref[...]
    t1 = jnp.dot(z, w1_ref[...], preferred_element_type=jnp.float32)
    w = jnp.dot(t1, w2_ref[...], preferred_element_type=jnp.float32)
    wt = lax.dot_general(
        w2_ref[...], t1, (((0,), (1,)), ((), ())),
        preferred_element_type=jnp.float32,
    )
    wt_ref[...] = wt
    fw_ref[...] = jnp.dot(w, gfw_ref[...], preferred_element_type=jnp.float32)


def _senet(z, w1, w2, gfw):
    nt = _B // _SENET_BT
    return pl.pallas_call(
        _senet_body,
        grid=(nt,),
        in_specs=[
            pl.BlockSpec((_SENET_BT, _F * _D0), lambda i: (i, 0)),
            pl.BlockSpec((_F * _D0, _F * _D0), lambda i: (0, 0)),
            pl.BlockSpec((_F * _D0, _K), lambda i: (0, 0)),
            pl.BlockSpec((_K, _F * (_F - 1)), lambda i: (0, 0)),
        ],
        out_specs=[
            pl.BlockSpec((_K, _SENET_BT), lambda i: (0, i)),
            pl.BlockSpec((_SENET_BT, _F * (_F - 1)), lambda i: (i, 0)),
        ],
        out_shape=[
            jax.ShapeDtypeStruct((_K, _B), jnp.float32),
            jax.ShapeDtypeStruct((_B, _F * (_F - 1)), jnp.float32),
        ],
    )(z, w1, w2, gfw)


_MERGE_BT = 128


_KC = 65
_BT4 = _MERGE_BT // 4


def _merge_body(g_ref, wt_ref, st_ref, e4_ref, out_ref):
    acc = jnp.zeros((_F, _MERGE_BT * _EMB), jnp.float32)
    for c in range(_K // _KC):
        g = g_ref[pl.ds(c * _KC, _KC)]
        w = wt_ref[pl.ds(c * _KC, _KC)]
        w4 = jnp.dot(w, e4_ref[...], preferred_element_type=jnp.float32)
        wg = g.reshape(_KC, _MERGE_BT * _EMB) * w4
        st = st_ref[:, pl.ds(c * _KC, _KC)]
        acc = acc + jnp.dot(
            st.astype(jnp.bfloat16),
            wg.astype(jnp.bfloat16),
            preferred_element_type=jnp.float32,
        )
    out_ref[...] = acc.reshape(_F, _BT4, 4 * _EMB)


def _merge(g4, wt, st, e4):
    nt = _B // _MERGE_BT
    return pl.pallas_call(
        _merge_body,
        grid=(nt,),
        in_specs=[
            pl.BlockSpec((_K, _BT4, 4 * _EMB), lambda j: (0, j, 0)),
            pl.BlockSpec((_K, _MERGE_BT), lambda j: (0, j)),
            pl.BlockSpec((_F, _K), lambda j: (0, 0)),
            pl.BlockSpec((_MERGE_BT, _MERGE_BT * _EMB), lambda j: (0, 0)),
        ],
        out_specs=pl.BlockSpec((_F, _BT4, 4 * _EMB), lambda j: (0, j, 0)),
        out_shape=jax.ShapeDtypeStruct((_F, _B // 4, 4 * _EMB), jnp.float32),
    )(g4, wt, st, e4)


def kernel(placeholder_inputs, origin_embeddings, codebook, senet_w1, senet_w2):
    g = _sc_gather(
        placeholder_inputs,
        jnp.asarray(_IK),
        jnp.asarray(_JK),
        jnp.asarray(_CK),
        codebook,
    )
    g4 = g.reshape(_K, _B // 4, 4 * _EMB)
    z = origin_embeddings.reshape(_B, _F * _D0)
    wt, fw = _senet(z, senet_w1, senet_w2, jnp.asarray(_GFW))
    out_t = _merge(g4, wt, jnp.asarray(_S_T), jnp.asarray(_E4))
    outputs = jnp.swapaxes(out_t.reshape(_F, _B, _EMB), 0, 1)
    field_weights = fw.reshape(_B, _F, _F - 1, 1)
    return outputs, field_weights


# speedup vs baseline: 1.8454x; 1.0728x over previous
import functools
import itertools

import jax
import jax.numpy as jnp
import numpy as np
from jax import lax
from jax.experimental import pallas as pl
from jax.experimental.pallas import tpu as pltpu
from jax.experimental.pallas import tpu_sc as plsc

_B = 4096
_F = 26
_D0 = 16
_EMB = 32
_NB = 1000000
_PAIRS = np.array(list(itertools.combinations(range(_F), 2)), dtype=np.int32)
_K = _PAIRS.shape[0]

_IK = _PAIRS[:, 0]
_JK = _PAIRS[:, 1]
_CK = (_IK.astype(np.int32) * 1822 + _JK.astype(np.int32) * 6649)

_F2I = np.zeros((_F, _F - 1), dtype=np.int32)
_cnt = np.zeros(_F, dtype=np.int32)
for _k, (_i, _j) in enumerate(_PAIRS):
    _F2I[_i, _cnt[_i]] = _k; _cnt[_i] += 1
    _F2I[_j, _cnt[_j]] = _k; _cnt[_j] += 1

_S_T = np.zeros((_F, _K), dtype=np.float32)
_S_T[_IK, np.arange(_K)] = 1.0
_S_T[_JK, np.arange(_K)] = 1.0

_GFW = np.zeros((_K, _F * (_F - 1)), dtype=np.float32)
_GFW[_F2I.reshape(-1), np.arange(_F * (_F - 1))] = 1.0

_E4 = np.zeros((128, 128 * 32), dtype=np.float32)
for _b in range(128):
    _E4[_b, (_b // 4) * 128 + (_b % 4) * 32 : (_b // 4) * 128 + (_b % 4) * 32 + 32] = 1.0

_NC = 2
_NS = 16
_NW = _NC * _NS
_BPW = _B // _NW
assert _BPW * _NW == _B


def _sc_gather(x, ik, jk, ck, codebook):
    mesh = plsc.VectorSubcoreMesh(core_axis_name="c", subcore_axis_name="s")

    @functools.partial(
        pl.kernel,
        out_type=jax.ShapeDtypeStruct((_K * _B, _EMB), jnp.float32),
        mesh=mesh,
        scratch_types=[
            pltpu.VMEM((_BPW, _F), jnp.int32),
            pltpu.VMEM((_K,), jnp.int32),
            pltpu.VMEM((_K,), jnp.int32),
            pltpu.VMEM((_K,), jnp.int32),
            pltpu.VMEM((4, _BPW), jnp.int32),
            pltpu.VMEM((4, _BPW, _EMB), jnp.float32),
            pltpu.SemaphoreType.DMA,
            pltpu.SemaphoreType.DMA,
        ],
        compiler_params=pltpu.CompilerParams(
            use_tc_tiling_on_sc=False, needs_layout_passes=False
        ),
    )
    def gather_kernel(x_hbm, ik_hbm, jk_hbm, ck_hbm, table_hbm, out_hbm,
                      x_v, ik_v, jk_v, ck_v, idx_v, rows_v, gsem, wsem):
        wid = lax.axis_index("s") * _NC + lax.axis_index("c")
        bbase = pl.multiple_of(wid * _BPW, _BPW)
        pltpu.sync_copy(x_hbm.at[pl.ds(bbase, _BPW), :], x_v)
        pltpu.sync_copy(ik_hbm, ik_v)
        pltpu.sync_copy(jk_hbm, jk_v)
        pltpu.sync_copy(ck_hbm, ck_v)

        lanes = jnp.arange(16, dtype=jnp.int32)

        def hash_chunk(c, slot):
            cvec = jnp.broadcast_to(c, (16,)).astype(jnp.int32)
            ikvec = plsc.load_gather(ik_v, [cvec])
            jkvec = plsc.load_gather(jk_v, [cvec])
            ckvec = plsc.load_gather(ck_v, [cvec])
            for li in range(_BPW // 16):
                bvec = lanes + (li * 16)
                xi = plsc.load_gather(x_v, [bvec, ikvec])
                xj = plsc.load_gather(x_v, [bvec, jkvec])
                h = xi * 40503 + xj * 7744 + ckvec
                r = lax.rem(h, _NB)
                r = jnp.where(r < 0, r + _NB, r)
                idx_v[slot, pl.ds(li * 16, 16)] = r

        def start(slot):
            pltpu.async_copy(
                table_hbm.at[idx_v.at[slot]], rows_v.at[slot], gsem
            )

        def out_at(c):
            orow = pl.multiple_of(c * _B + wid * _BPW, _BPW)
            return out_hbm.at[pl.ds(orow, _BPW), :]

        def body(c, carry):
            slot = lax.rem(c, 4)
            pslot = lax.rem(c + 2, 4)

            @pl.when(c < _K)
            def _launch():
                hash_chunk(c, slot)

                @pl.when(c >= 4)
                def _drain_write():
                    pltpu.make_async_copy(
                        rows_v.at[slot], out_at(c - 4), wsem
                    ).wait()

                start(slot)

            @pl.when(c >= 2)
            def _emit():
                pltpu.make_async_copy(
                    table_hbm.at[idx_v.at[pslot]], rows_v.at[pslot], gsem
                ).wait()
                pltpu.async_copy(rows_v.at[pslot], out_at(c - 2), wsem)

            return carry

        lax.fori_loop(0, _K + 2, body, 0)

        for t in range(1, 5):
            pltpu.make_async_copy(
                rows_v.at[(_K - t) % 4], out_at(_K - t), wsem
            ).wait()

    return gather_kernel(x, ik, jk, ck, codebook)


_SENET_BT = 256


def _senet_body(z_ref, w1_ref, w2_ref, gfw_ref, wt_ref, fw_ref):
    z = z_ref[...]
    t1 = jnp.dot(z, w1_ref[...], preferred_element_type=jnp.float32)
    w = jnp.dot(t1, w2_ref[...], preferred_element_type=jnp.float32)
    wt = lax.dot_general(
        w2_ref[...], t1, (((0,), (1,)), ((), ())),
        preferred_element_type=jnp.float32,
    )
    wt_ref[...] = wt
    fw_ref[...] = jnp.dot(w, gfw_ref[...], preferred_element_type=jnp.float32)


def _senet(z, w1, w2, gfw):
    nt = _B // _SENET_BT
    return pl.pallas_call(
        _senet_body,
        grid=(nt,),
        in_specs=[
            pl.BlockSpec((_SENET_BT, _F * _D0), lambda i: (i, 0)),
            pl.BlockSpec((_F * _D0, _F * _D0), lambda i: (0, 0)),
            pl.BlockSpec((_F * _D0, _K), lambda i: (0, 0)),
            pl.BlockSpec((_K, _F * (_F - 1)), lambda i: (0, 0)),
        ],
        out_specs=[
            pl.BlockSpec((_K, _SENET_BT), lambda i: (0, i)),
            pl.BlockSpec((_SENET_BT, _F * (_F - 1)), lambda i: (i, 0)),
        ],
        out_shape=[
            jax.ShapeDtypeStruct((_K, _B), jnp.float32),
            jax.ShapeDtypeStruct((_B, _F * (_F - 1)), jnp.float32),
        ],
    )(z, w1, w2, gfw)


_MERGE_BT = 128


_KC = 65
_BT4 = _MERGE_BT // 4


def _merge_body(g_ref, wt_ref, st_ref, e4_ref, out_ref):
    acc = jnp.zeros((_F, _MERGE_BT * _EMB), jnp.float32)
    for c in range(_K // _KC):
        g = g_ref[pl.ds(c * _KC, _KC)]
        w = wt_ref[pl.ds(c * _KC, _KC)]
        w4 = jnp.dot(w, e4_ref[...], preferred_element_type=jnp.float32)
        wg = g.reshape(_KC, _MERGE_BT * _EMB) * w4
        st = st_ref[:, pl.ds(c * _KC, _KC)]
        acc = acc + jnp.dot(
            st.astype(jnp.bfloat16),
            wg.astype(jnp.bfloat16),
            preferred_element_type=jnp.float32,
        )
    out_ref[...] = acc.reshape(_F, _BT4, 4 * _EMB)


def _merge(g4, wt, st, e4):
    nt = _B // _MERGE_BT
    return pl.pallas_call(
        _merge_body,
        grid=(nt,),
        in_specs=[
            pl.BlockSpec((_K, _BT4, 4 * _EMB), lambda j: (0, j, 0)),
            pl.BlockSpec((_K, _MERGE_BT), lambda j: (0, j)),
            pl.BlockSpec((_F, _K), lambda j: (0, 0)),
            pl.BlockSpec((_MERGE_BT, _MERGE_BT * _EMB), lambda j: (0, 0)),
        ],
        out_specs=pl.BlockSpec((_F, _BT4, 4 * _EMB), lambda j: (0, j, 0)),
        out_shape=jax.ShapeDtypeStruct((_F, _B // 4, 4 * _EMB), jnp.float32),
    )(g4, wt, st, e4)


def kernel(placeholder_inputs, origin_embeddings, codebook, senet_w1, senet_w2):
    g = _sc_gather(
        placeholder_inputs,
        jnp.asarray(_IK),
        jnp.asarray(_JK),
        jnp.asarray(_CK),
        codebook,
    )
    g4 = g.reshape(_K, _B // 4, 4 * _EMB)
    z = origin_embeddings.reshape(_B, _F * _D0)
    wt, fw = _senet(z, senet_w1, senet_w2, jnp.asarray(_GFW))
    out_t = _merge(g4, wt, jnp.asarray(_S_T), jnp.asarray(_E4))
    outputs = jnp.swapaxes(out_t.reshape(_F, _B, _EMB), 0, 1)
    field_weights = fw.reshape(_B, _F, _F - 1, 1)
    return outputs, field_weights


# speedup vs baseline: 1.8656x; 1.0109x over previous
import functools
import itertools

import jax
import jax.numpy as jnp
import numpy as np
from jax import lax
from jax.experimental import pallas as pl
from jax.experimental.pallas import tpu as pltpu
from jax.experimental.pallas import tpu_sc as plsc

_B = 4096
_F = 26
_D0 = 16
_EMB = 32
_NB = 1000000
_PAIRS = np.array(list(itertools.combinations(range(_F), 2)), dtype=np.int32)
_K = _PAIRS.shape[0]

_IK = _PAIRS[:, 0]
_JK = _PAIRS[:, 1]
_CK = (_IK.astype(np.int32) * 1822 + _JK.astype(np.int32) * 6649)

_F2I = np.zeros((_F, _F - 1), dtype=np.int32)
_cnt = np.zeros(_F, dtype=np.int32)
for _k, (_i, _j) in enumerate(_PAIRS):
    _F2I[_i, _cnt[_i]] = _k; _cnt[_i] += 1
    _F2I[_j, _cnt[_j]] = _k; _cnt[_j] += 1

_S_T = np.zeros((_F, _K), dtype=np.float32)
_S_T[_IK, np.arange(_K)] = 1.0
_S_T[_JK, np.arange(_K)] = 1.0

_GFW = np.zeros((_K, _F * (_F - 1)), dtype=np.float32)
_GFW[_F2I.reshape(-1), np.arange(_F * (_F - 1))] = 1.0

_E4 = np.zeros((128, 128 * 32), dtype=np.float32)
for _b in range(128):
    _E4[_b, (_b // 4) * 128 + (_b % 4) * 32 : (_b // 4) * 128 + (_b % 4) * 32 + 32] = 1.0

_NC = 2
_NS = 16
_NW = _NC * _NS
_BPW = _B // _NW
assert _BPW * _NW == _B


def _sc_gather(x, ik, jk, ck, codebook):
    mesh = plsc.VectorSubcoreMesh(core_axis_name="c", subcore_axis_name="s")

    @functools.partial(
        pl.kernel,
        out_type=jax.ShapeDtypeStruct((_K * _B, _EMB), jnp.float32),
        mesh=mesh,
        scratch_types=[
            pltpu.VMEM((_BPW, _F), jnp.int32),
            pltpu.VMEM((_K,), jnp.int32),
            pltpu.VMEM((_K,), jnp.int32),
            pltpu.VMEM((_K,), jnp.int32),
            pltpu.VMEM((8, _BPW), jnp.int32),
            pltpu.VMEM((8, _BPW, _EMB), jnp.float32),
            pltpu.SemaphoreType.DMA,
            pltpu.SemaphoreType.DMA,
        ],
        compiler_params=pltpu.CompilerParams(
            use_tc_tiling_on_sc=False, needs_layout_passes=False
        ),
    )
    def gather_kernel(x_hbm, ik_hbm, jk_hbm, ck_hbm, table_hbm, out_hbm,
                      x_v, ik_v, jk_v, ck_v, idx_v, rows_v, gsem, wsem):
        wid = lax.axis_index("s") * _NC + lax.axis_index("c")
        bbase = pl.multiple_of(wid * _BPW, _BPW)
        pltpu.sync_copy(x_hbm.at[pl.ds(bbase, _BPW), :], x_v)
        pltpu.sync_copy(ik_hbm, ik_v)
        pltpu.sync_copy(jk_hbm, jk_v)
        pltpu.sync_copy(ck_hbm, ck_v)

        lanes = jnp.arange(16, dtype=jnp.int32)

        def hash_chunk(c, slot):
            cvec = jnp.broadcast_to(c, (16,)).astype(jnp.int32)
            ikvec = plsc.load_gather(ik_v, [cvec])
            jkvec = plsc.load_gather(jk_v, [cvec])
            ckvec = plsc.load_gather(ck_v, [cvec])
            for li in range(_BPW // 16):
                bvec = lanes + (li * 16)
                xi = plsc.load_gather(x_v, [bvec, ikvec])
                xj = plsc.load_gather(x_v, [bvec, jkvec])
                h = xi * 40503 + xj * 7744 + ckvec
                r = lax.rem(h, _NB)
                r = jnp.where(r < 0, r + _NB, r)
                idx_v[slot, pl.ds(li * 16, 16)] = r

        def start(slot):
            pltpu.async_copy(
                table_hbm.at[idx_v.at[slot]], rows_v.at[slot], gsem
            )

        def out_at(c):
            orow = pl.multiple_of(c * _B + wid * _BPW, _BPW)
            return out_hbm.at[pl.ds(orow, _BPW), :]

        def body(c, carry):
            slot = lax.rem(c, 8)
            pslot = lax.rem(c + 5, 8)

            @pl.when(c < _K)
            def _launch():
                hash_chunk(c, slot)

                @pl.when(c >= 8)
                def _drain_write():
                    pltpu.make_async_copy(
                        rows_v.at[slot], out_at(c - 8), wsem
                    ).wait()

                start(slot)

            @pl.when(c >= 3)
            def _emit():
                pltpu.make_async_copy(
                    table_hbm.at[idx_v.at[pslot]], rows_v.at[pslot], gsem
                ).wait()
                pltpu.async_copy(rows_v.at[pslot], out_at(c - 3), wsem)

            return carry

        lax.fori_loop(0, _K + 3, body, 0)

        for t in range(1, 9):
            pltpu.make_async_copy(
                rows_v.at[(_K - t) % 8], out_at(_K - t), wsem
            ).wait()

    return gather_kernel(x, ik, jk, ck, codebook)


_SENET_BT = 256


def _senet_body(z_ref, w1_ref, w2_ref, gfw_ref, wt_ref, fw_ref):
    z = z_ref[...]
    t1 = jnp.dot(z, w1_ref[...], preferred_element_type=jnp.float32)
    w = jnp.dot(t1, w2_ref[...], preferred_element_type=jnp.float32)
    wt = lax.dot_general(
        w2_ref[...], t1, (((0,), (1,)), ((), ())),
        preferred_element_type=jnp.float32,
    )
    wt_ref[...] = wt
    fw_ref[...] = jnp.dot(w, gfw_ref[...], preferred_element_type=jnp.float32)


def _senet(z, w1, w2, gfw):
    nt = _B // _SENET_BT
    return pl.pallas_call(
        _senet_body,
        grid=(nt,),
        in_specs=[
            pl.BlockSpec((_SENET_BT, _F * _D0), lambda i: (i, 0)),
            pl.BlockSpec((_F * _D0, _F * _D0), lambda i: (0, 0)),
            pl.BlockSpec((_F * _D0, _K), lambda i: (0, 0)),
            pl.BlockSpec((_K, _F * (_F - 1)), lambda i: (0, 0)),
        ],
        out_specs=[
            pl.BlockSpec((_K, _SENET_BT), lambda i: (0, i)),
            pl.BlockSpec((_SENET_BT, _F * (_F - 1)), lambda i: (i, 0)),
        ],
        out_shape=[
            jax.ShapeDtypeStruct((_K, _B), jnp.float32),
            jax.ShapeDtypeStruct((_B, _F * (_F - 1)), jnp.float32),
        ],
    )(z, w1, w2, gfw)


_MERGE_BT = 128


_KC = 65
_BT4 = _MERGE_BT // 4


def _merge_body(g_ref, wt_ref, st_ref, e4_ref, out_ref):
    acc = jnp.zeros((_F, _MERGE_BT * _EMB), jnp.float32)
    for c in range(_K // _KC):
        g = g_ref[pl.ds(c * _KC, _KC)]
        w = wt_ref[pl.ds(c * _KC, _KC)]
        w4 = jnp.dot(w, e4_ref[...], preferred_element_type=jnp.float32)
        wg = g.reshape(_KC, _MERGE_BT * _EMB) * w4
        st = st_ref[:, pl.ds(c * _KC, _KC)]
        acc = acc + jnp.dot(
            st.astype(jnp.bfloat16),
            wg.astype(jnp.bfloat16),
            preferred_element_type=jnp.float32,
        )
    out_ref[...] = acc.reshape(_F, _BT4, 4 * _EMB)


def _merge(g4, wt, st, e4):
    nt = _B // _MERGE_BT
    return pl.pallas_call(
        _merge_body,
        grid=(nt,),
        in_specs=[
            pl.BlockSpec((_K, _BT4, 4 * _EMB), lambda j: (0, j, 0)),
            pl.BlockSpec((_K, _MERGE_BT), lambda j: (0, j)),
            pl.BlockSpec((_F, _K), lambda j: (0, 0)),
            pl.BlockSpec((_MERGE_BT, _MERGE_BT * _EMB), lambda j: (0, 0)),
        ],
        out_specs=pl.BlockSpec((_F, _BT4, 4 * _EMB), lambda j: (0, j, 0)),
        out_shape=jax.ShapeDtypeStruct((_F, _B // 4, 4 * _EMB), jnp.float32),
    )(g4, wt, st, e4)


def kernel(placeholder_inputs, origin_embeddings, codebook, senet_w1, senet_w2):
    g = _sc_gather(
        placeholder_inputs,
        jnp.asarray(_IK),
        jnp.asarray(_JK),
        jnp.asarray(_CK),
        codebook,
    )
    g4 = g.reshape(_K, _B // 4, 4 * _EMB)
    z = origin_embeddings.reshape(_B, _F * _D0)
    wt, fw = _senet(z, senet_w1, senet_w2, jnp.asarray(_GFW))
    out_t = _merge(g4, wt, jnp.asarray(_S_T), jnp.asarray(_E4))
    outputs = jnp.swapaxes(out_t.reshape(_F, _B, _EMB), 0, 1)
    field_weights = fw.reshape(_B, _F, _F - 1, 1)
    return outputs, field_weights
